# Initial kernel scaffold; baseline (speedup 1.0000x reference)
import jax
import jax.numpy as jnp
from jax.experimental import pallas as pl


def kernel(x, tok_table, pos_table):
    raise NotImplementedError("write your pallas kernel here")


# speedup vs baseline: 1.1917x; 1.1917x over previous
import functools

import jax
import jax.numpy as jnp
from jax import lax
from jax.experimental import pallas as pl
from jax.experimental.pallas import tpu as pltpu
from jax.experimental.pallas import tpu_sc as plsc

_LANES = 16
_IDX_PER_GATHER = 128


def _build(B, S, E, V):
    info = plsc.get_sparse_core_info()
    NC, NS = info.num_cores, info.num_subcores
    NW = NC * NS
    total = B * S
    assert total % NW == 0
    per_w = total // NW
    n_gather = 10
    chunk = n_gather * _IDX_PER_GATHER
    assert per_w % chunk == 0
    n_chunk = per_w // chunk
    assert per_w % S == 0

    mesh = plsc.VectorSubcoreMesh(core_axis_name="c", subcore_axis_name="s")

    @functools.partial(
        pl.kernel,
        mesh=mesh,
        compiler_params=pltpu.CompilerParams(use_tc_tiling_on_sc=False),
        out_type=jax.ShapeDtypeStruct((NW, n_chunk, chunk, E), jnp.float32),
        scratch_types=[
            pltpu.VMEM((n_gather, _IDX_PER_GATHER), jnp.int32),
            pltpu.VMEM((chunk, E), jnp.float32),
            pltpu.VMEM((S, E), jnp.float32),
            pltpu.SemaphoreType.DMA,
        ],
    )
    def k(x_hbm, tok_hbm, pos_hbm, out_hbm, idx_v, rows_v, pos_v, sem):
        wid = lax.axis_index("s") * NC + lax.axis_index("c")
        pltpu.sync_copy(pos_hbm, pos_v)

        def chunk_body(c, _):
            pltpu.sync_copy(x_hbm.at[wid, c], idx_v)
            copies = [
                pltpu.async_copy(
                    tok_hbm.at[idx_v.at[j]],
                    rows_v.at[pl.ds(j * _IDX_PER_GATHER, _IDX_PER_GATHER)],
                    sem,
                )
                for j in range(n_gather)
            ]
            for cp in copies:
                cp.wait()

            def add_body(i, _):
                r = (c * chunk + i) % S
                for h in range(0, E, _LANES):
                    rows_v[i, pl.ds(h, _LANES)] = (
                        rows_v[i, pl.ds(h, _LANES)] + pos_v[r, pl.ds(h, _LANES)]
                    )
                return 0

            lax.fori_loop(0, chunk, add_body, 0)
            pltpu.sync_copy(rows_v, out_hbm.at[wid, c])
            return 0

        lax.fori_loop(0, n_chunk, chunk_body, 0)

    return k, NW, n_chunk, chunk, n_gather


def kernel(x, tok_table, pos_table):
    B, S = x.shape
    V, E = tok_table.shape
    k, NW, n_chunk, chunk, n_gather = _build(B, S, E, V)
    xr = x.astype(jnp.int32).reshape(NW, n_chunk, n_gather, _IDX_PER_GATHER)
    out = k(xr, tok_table, pos_table)
    return out.reshape(B, S, E)


# speedup vs baseline: 1.4847x; 1.2459x over previous
import functools

import jax
import jax.numpy as jnp
from jax import lax
from jax.experimental import pallas as pl
from jax.experimental.pallas import tpu as pltpu
from jax.experimental.pallas import tpu_sc as plsc

_LANES = 16


def _build(B, S, E, V):
    info = plsc.get_sparse_core_info()
    NC, NS = info.num_cores, info.num_subcores
    NW = NC * NS
    assert B % NW == 0
    rows_w = B // NW
    RC = 8
    n_chunk = rows_w // RC
    assert n_chunk % 2 == 0
    g0 = 128
    g1 = S - g0

    mesh = plsc.VectorSubcoreMesh(core_axis_name="c", subcore_axis_name="s")

    @functools.partial(
        pl.kernel,
        mesh=mesh,
        compiler_params=pltpu.CompilerParams(use_tc_tiling_on_sc=False),
        out_type=jax.ShapeDtypeStruct((B, S, E), jnp.float32),
        scratch_types=[
            pltpu.VMEM((RC, S), jnp.int32),
            pltpu.VMEM((RC, S), jnp.int32),
            pltpu.VMEM((RC, S, E), jnp.float32),
            pltpu.VMEM((RC, S, E), jnp.float32),
            pltpu.VMEM((S, E), jnp.float32),
            pltpu.SemaphoreType.DMA,
            pltpu.SemaphoreType.DMA,
            pltpu.SemaphoreType.DMA,
            pltpu.SemaphoreType.DMA,
        ],
    )
    def k(x_hbm, tok_hbm, pos_hbm, out_hbm,
          idx0, idx1, rows0, rows1, pos_v,
          sem_g0, sem_g1, sem_o0, sem_o1):
        wid = lax.axis_index("s") * NC + lax.axis_index("c")
        row0 = wid * rows_w
        pltpu.sync_copy(pos_hbm, pos_v)

        def load_idx(idx_v, c):
            pltpu.sync_copy(x_hbm.at[pl.ds(row0 + c * RC, RC)], idx_v)

        def fire_gathers(idx_v, rows_v, sem):
            for b in range(RC):
                pltpu.async_copy(
                    tok_hbm.at[idx_v.at[b, pl.ds(0, g0)]],
                    rows_v.at[b, pl.ds(0, g0)], sem)
                pltpu.async_copy(
                    tok_hbm.at[idx_v.at[b, pl.ds(g0, g1)]],
                    rows_v.at[b, pl.ds(g0, g1)], sem)

        def drain(vmem_ref, sem):
            pltpu.make_async_copy(out_hbm.at[pl.ds(0, RC)], vmem_ref, sem).wait()

        def add_pos(rows_v):
            def body(s, _):
                p0 = pos_v[s, pl.ds(0, _LANES)]
                p1 = pos_v[s, pl.ds(_LANES, _LANES)]
                for b in range(RC):
                    rows_v[b, s, pl.ds(0, _LANES)] = (
                        rows_v[b, s, pl.ds(0, _LANES)] + p0)
                    rows_v[b, s, pl.ds(_LANES, _LANES)] = (
                        rows_v[b, s, pl.ds(_LANES, _LANES)] + p1)
                return 0
            lax.fori_loop(0, S, body, 0)

        def put_out(rows_v, c, sem):
            pltpu.async_copy(rows_v, out_hbm.at[pl.ds(row0 + c * RC, RC)], sem)

        load_idx(idx0, 0)
        fire_gathers(idx0, rows0, sem_g0)

        def pair_body(c2, _):
            cA = 2 * c2

            @pl.when(c2 > 0)
            def _():
                drain(rows1, sem_o1)
            load_idx(idx1, cA + 1)
            fire_gathers(idx1, rows1, sem_g1)
            drain(rows0, sem_g0)
            add_pos(rows0)
            put_out(rows0, cA, sem_o0)

            @pl.when(c2 < n_chunk // 2 - 1)
            def _():
                drain(rows0, sem_o0)
                load_idx(idx0, cA + 2)
                fire_gathers(idx0, rows0, sem_g0)
            drain(rows1, sem_g1)
            add_pos(rows1)
            put_out(rows1, cA + 1, sem_o1)
            return 0

        lax.fori_loop(0, n_chunk // 2, pair_body, 0)

        drain(rows0, sem_o0)
        drain(rows1, sem_o1)

    return k


def kernel(x, tok_table, pos_table):
    B, S = x.shape
    V, E = tok_table.shape
    k = _build(B, S, E, V)
    return k(x.astype(jnp.int32), tok_table, pos_table)
